# baseline (speedup 1.0000x reference)
import functools

import jax
import jax.numpy as jnp
from jax.experimental import pallas as pl
from jax.experimental.pallas import tpu as pltpu
from jax.experimental.pallas import tpu_sc as plsc

B, N = 4096, 200
TOTAL = B * N
EMB_DIM = 32
NUM_DIM = 16
OUT_DIM = 64

PACK = 128 // EMB_DIM
PR = B // PACK
P_BLK = 8
SLICES = 5
NS = N // SLICES


def _sc_gather_packed(emb_table, idx_2d):
    mesh = plsc.VectorSubcoreMesh(core_axis_name="c", subcore_axis_name="s")

    @functools.partial(
        pl.kernel,
        out_type=jax.ShapeDtypeStruct((NS * PR, 128), jnp.float32),
        mesh=mesh,
        compiler_params=pltpu.CompilerParams(use_tc_tiling_on_sc=False),
    )
    def gather_kernel(table_hbm, idx_hbm, out_hbm):
        def body(i_vmem, o_vmem):
            pltpu.sync_copy(table_hbm.at[i_vmem.at[0]], o_vmem)

        SUB = 2
        C = PR // SUB
        pltpu.emit_pipeline(
            body,
            grid=(NS * PACK * SUB,),
            in_specs=[pl.BlockSpec((1, C), lambda t: (0, t))],
            out_specs=[pl.BlockSpec(
                (C, EMB_DIM),
                lambda t: ((t // (PACK * SUB)) * SUB + t % SUB,
                           (t // SUB) % PACK))],
            core_axis_name=("c", "s"),
            dimension_semantics=(pltpu.PARALLEL,),
        )(idx_hbm, out_hbm)

    return gather_kernel(emb_table, idx_2d)


def _tc_project_t(packed, n_t, Wm, b_col):
    def body(e_ref, n_ref, w_ref, b_ref, o_ref):
        for q in range(P_BLK):
            accn = jax.lax.dot_general(
                w_ref[:, EMB_DIM:], n_ref[q],
                (((1,), (0,)), ((), ())),
                preferred_element_type=jnp.float32)
            for j in range(PACK):
                g = e_ref[pl.ds(q * PR, PR), pl.ds(j * EMB_DIM, EMB_DIM)]
                og = jax.lax.dot_general(
                    w_ref[:, :EMB_DIM], g,
                    (((1,), (1,)), ((), ())),
                    preferred_element_type=jnp.float32)
                a = jax.lax.slice(accn, (0, j * PR), (OUT_DIM, (j + 1) * PR))
                o_ref[q, :, pl.ds(j * PR, PR)] = jnp.maximum(
                    og + a + b_ref[...], 0.0)

    return pl.pallas_call(
        body,
        grid=(NS // P_BLK,),
        in_specs=[
            pl.BlockSpec((P_BLK * PR, 128), lambda i: (i, 0)),
            pl.BlockSpec((P_BLK, NUM_DIM, B), lambda i: (i, 0, 0)),
            pl.BlockSpec((OUT_DIM, EMB_DIM + NUM_DIM), lambda i: (0, 0)),
            pl.BlockSpec((OUT_DIM, 1), lambda i: (0, 0)),
        ],
        out_specs=pl.BlockSpec((P_BLK, OUT_DIM, B), lambda i: (i, 0, 0)),
        out_shape=jax.ShapeDtypeStruct((NS, OUT_DIM, B), jnp.float32),
    )(packed, n_t, Wm, b_col)


def kernel(unit_types, numeric, emb_table, W, b):
    idx_2d = unit_types.astype(jnp.int32).T.reshape(1, TOTAL)
    n_t = jnp.transpose(numeric, (1, 2, 0))
    b_col = b.reshape(OUT_DIM, 1)
    outs = []
    for s in range(SLICES):
        idx_s = jax.lax.slice(idx_2d, (0, s * NS * B), (1, (s + 1) * NS * B))
        packed = _sc_gather_packed(emb_table, idx_s)
        n_s = jax.lax.slice(n_t, (s * NS, 0, 0), ((s + 1) * NS, NUM_DIM, B))
        outs.append(_tc_project_t(packed, n_s, W, b_col))
    out_t = jnp.concatenate(outs, axis=0)
    return jnp.transpose(out_t, (2, 0, 1))


# speedup vs baseline: 1.6005x; 1.6005x over previous
import functools

import jax
import jax.numpy as jnp
from jax.experimental import pallas as pl
from jax.experimental.pallas import tpu as pltpu
from jax.experimental.pallas import tpu_sc as plsc

B, N = 4096, 200
TOTAL = B * N
EMB_DIM = 32
NUM_DIM = 16
OUT_DIM = 64

PACK = 128 // EMB_DIM
PR = B // PACK
P_BLK = 8
SLICES = 5
NS = N // SLICES
SUB = 2
C = PR // SUB
TPS = NS * PACK * SUB


def _sc_gather_packed(emb_table, idx_2d, s):
    mesh = plsc.VectorSubcoreMesh(core_axis_name="c", subcore_axis_name="s")
    off = s * TPS

    @functools.partial(
        pl.kernel,
        out_type=jax.ShapeDtypeStruct((NS * PR, 128), jnp.float32),
        mesh=mesh,
        compiler_params=pltpu.CompilerParams(use_tc_tiling_on_sc=False),
    )
    def gather_kernel(table_hbm, idx_hbm, out_hbm):
        def body(i_vmem, o_vmem):
            pltpu.sync_copy(table_hbm.at[i_vmem.at[0]], o_vmem)

        pltpu.emit_pipeline(
            body,
            grid=(TPS,),
            in_specs=[pl.BlockSpec((1, C), lambda t: (0, off + t))],
            out_specs=[pl.BlockSpec(
                (C, EMB_DIM),
                lambda t: ((t // (PACK * SUB)) * SUB + t % SUB,
                           (t // SUB) % PACK))],
            core_axis_name=("c", "s"),
            dimension_semantics=(pltpu.PARALLEL,),
        )(idx_hbm, out_hbm)

    return gather_kernel(emb_table, idx_2d)


def _tc_project_t(packed, n_t, Wm, b_col, s, prev):
    boff = s * (NS // P_BLK)

    def body(e_ref, n_ref, w_ref, b_ref, *rest):
        o_ref = rest[-1]
        for q in range(P_BLK):
            accn = jax.lax.dot_general(
                w_ref[:, EMB_DIM:], n_ref[q],
                (((1,), (0,)), ((), ())),
                preferred_element_type=jnp.float32)
            for j in range(PACK):
                g = e_ref[pl.ds(q * PR, PR), pl.ds(j * EMB_DIM, EMB_DIM)]
                og = jax.lax.dot_general(
                    w_ref[:, :EMB_DIM], g,
                    (((1,), (1,)), ((), ())),
                    preferred_element_type=jnp.float32)
                a = jax.lax.slice(accn, (0, j * PR), (OUT_DIM, (j + 1) * PR))
                o_ref[q, :, pl.ds(j * PR, PR)] = jnp.maximum(
                    og + a + b_ref[...], 0.0)

    in_specs = [
        pl.BlockSpec((P_BLK * PR, 128), lambda i: (i, 0)),
        pl.BlockSpec((P_BLK, NUM_DIM, B), lambda i: (boff + i, 0, 0)),
        pl.BlockSpec((OUT_DIM, EMB_DIM + NUM_DIM), lambda i: (0, 0)),
        pl.BlockSpec((OUT_DIM, 1), lambda i: (0, 0)),
    ]
    args = [packed, n_t, Wm, b_col]
    aliases = {}
    if prev is not None:
        in_specs.append(pl.BlockSpec(memory_space=pl.ANY))
        args.append(prev)
        aliases = {4: 0}

    return pl.pallas_call(
        body,
        grid=(NS // P_BLK,),
        in_specs=in_specs,
        out_specs=pl.BlockSpec((P_BLK, OUT_DIM, B),
                               lambda i: (boff + i, 0, 0)),
        out_shape=jax.ShapeDtypeStruct((N, OUT_DIM, B), jnp.float32),
        input_output_aliases=aliases,
    )(*args)


def kernel(unit_types, numeric, emb_table, W, b):
    idx_2d = unit_types.astype(jnp.int32).T.reshape(1, TOTAL)
    n_t = jnp.transpose(numeric, (1, 2, 0))
    b_col = b.reshape(OUT_DIM, 1)
    out_t = None
    for s in range(SLICES):
        packed = _sc_gather_packed(emb_table, idx_2d, s)
        out_t = _tc_project_t(packed, n_t, W, b_col, s, out_t)
    return jnp.transpose(out_t, (2, 0, 1))


# speedup vs baseline: 1.6131x; 1.0079x over previous
import functools

import jax
import jax.numpy as jnp
from jax.experimental import pallas as pl
from jax.experimental.pallas import tpu as pltpu
from jax.experimental.pallas import tpu_sc as plsc

B, N = 4096, 200
TOTAL = B * N
EMB_DIM = 32
NUM_DIM = 16
OUT_DIM = 64

PACK = 128 // EMB_DIM
PR = B // PACK
P_BLK = 10
SLICES = 2
NS = N // SLICES
SUB = 2
C = PR // SUB
TPS = NS * PACK * SUB


def _sc_gather_packed(emb_table, idx_2d, s):
    mesh = plsc.VectorSubcoreMesh(core_axis_name="c", subcore_axis_name="s")
    off = s * TPS

    @functools.partial(
        pl.kernel,
        out_type=jax.ShapeDtypeStruct((NS * PR, 128), jnp.float32),
        mesh=mesh,
        compiler_params=pltpu.CompilerParams(use_tc_tiling_on_sc=False),
    )
    def gather_kernel(table_hbm, idx_hbm, out_hbm):
        def body(i_vmem, o_vmem):
            pltpu.sync_copy(table_hbm.at[i_vmem.at[0]], o_vmem)

        pltpu.emit_pipeline(
            body,
            grid=(TPS,),
            in_specs=[pl.BlockSpec((1, C), lambda t: (0, off + t))],
            out_specs=[pl.BlockSpec(
                (C, EMB_DIM),
                lambda t: ((t // (PACK * SUB)) * SUB + t % SUB,
                           (t // SUB) % PACK))],
            core_axis_name=("c", "s"),
            dimension_semantics=(pltpu.PARALLEL,),
        )(idx_hbm, out_hbm)

    return gather_kernel(emb_table, idx_2d)


def _tc_project_t(packed, n_t, Wm, b_col, s, prev):
    boff = s * (NS // P_BLK)

    def body(e_ref, n_ref, w_ref, b_ref, *rest):
        o_ref = rest[-1]
        for q in range(P_BLK):
            accn = jax.lax.dot_general(
                w_ref[:, EMB_DIM:], n_ref[q],
                (((1,), (0,)), ((), ())),
                preferred_element_type=jnp.float32)
            for j in range(PACK):
                g = e_ref[pl.ds(q * PR, PR), pl.ds(j * EMB_DIM, EMB_DIM)]
                og = jax.lax.dot_general(
                    w_ref[:, :EMB_DIM], g,
                    (((1,), (1,)), ((), ())),
                    preferred_element_type=jnp.float32)
                a = jax.lax.slice(accn, (0, j * PR), (OUT_DIM, (j + 1) * PR))
                o_ref[q, :, pl.ds(j * PR, PR)] = jnp.maximum(
                    og + a + b_ref[...], 0.0)

    in_specs = [
        pl.BlockSpec((P_BLK * PR, 128), lambda i: (i, 0)),
        pl.BlockSpec((P_BLK, NUM_DIM, B), lambda i: (boff + i, 0, 0)),
        pl.BlockSpec((OUT_DIM, EMB_DIM + NUM_DIM), lambda i: (0, 0)),
        pl.BlockSpec((OUT_DIM, 1), lambda i: (0, 0)),
    ]
    args = [packed, n_t, Wm, b_col]
    aliases = {}
    if prev is not None:
        in_specs.append(pl.BlockSpec(memory_space=pl.ANY))
        args.append(prev)
        aliases = {4: 0}

    return pl.pallas_call(
        body,
        grid=(NS // P_BLK,),
        in_specs=in_specs,
        out_specs=pl.BlockSpec((P_BLK, OUT_DIM, B),
                               lambda i: (boff + i, 0, 0)),
        out_shape=jax.ShapeDtypeStruct((N, OUT_DIM, B), jnp.float32),
        input_output_aliases=aliases,
    )(*args)


def kernel(unit_types, numeric, emb_table, W, b):
    idx_2d = unit_types.astype(jnp.int32).T.reshape(1, TOTAL)
    n_t = jnp.transpose(numeric, (1, 2, 0))
    b_col = b.reshape(OUT_DIM, 1)
    out_t = None
    for s in range(SLICES):
        packed = _sc_gather_packed(emb_table, idx_2d, s)
        out_t = _tc_project_t(packed, n_t, W, b_col, s, out_t)
    return jnp.transpose(out_t, (2, 0, 1))
